# baseline (speedup 1.0000x reference)
import functools

import jax
import jax.numpy as jnp
from jax import lax
from jax.experimental import pallas as pl
from jax.experimental.pallas import tpu as pltpu
from jax.experimental.pallas import tpu_sc as plsc

N = 10000
E = 320000
NC = 2
NS = 16
L = 16
CH = 64
RW = 128
EPT = E // NS
EP = 2 * CH * (-(-EPT // (2 * CH)))
EPAD = NS * EP
NCH = EP // CH
NRCH = -(-N // CH)
NRQ = -(-NRCH // NS)
LASTR = N - (NRCH - 1) * CH


def _sc_mesh():
    return plsc.VectorSubcoreMesh(
        core_axis_name="c", subcore_axis_name="s", num_cores=NC, num_subcores=NS
    )


def _for_node_chunks(s, fn):
    for q in range(NRQ):
        cq = s + NS * q
        row0 = pl.multiple_of(cq * CH, 8)

        @pl.when(cq < NRCH - 1)
        def _full():
            fn(row0, CH)

        @pl.when(cq == NRCH - 1)
        def _tail():
            fn(row0, LASTR)


def _degrees(didx4, ew16, x):
    @functools.partial(
        pl.kernel,
        out_type=[
            jax.ShapeDtypeStruct((NC, N, RW), jnp.float32),
            jax.ShapeDtypeStruct((2 * N, RW), jnp.float32),
        ],
        mesh=_sc_mesh(),
        scratch_types=[
            pltpu.VMEM((CH,), jnp.int32),
            pltpu.VMEM((CH,), jnp.int32),
            pltpu.VMEM((CH, L), jnp.float32),
            pltpu.VMEM((CH, L), jnp.float32),
            pltpu.VMEM((CH, RW), jnp.float32),
            pltpu.VMEM((CH, RW), jnp.float32),
            pltpu.VMEM_SHARED((N, RW), jnp.float32),
            pltpu.SemaphoreType.DMA,
            pltpu.SemaphoreType.DMA,
            pltpu.SemaphoreType.DMA,
            pltpu.SemaphoreType.DMA,
            pltpu.SemaphoreType.DMA,
            pltpu.SemaphoreType.DMA,
        ],
    )
    def body(didx_h, ew16_h, x_h, out_h, t1_h, di0, di1, ewr0, ewr1, stage0,
             stage1, acc_sh, dsem0, dsem1, esem0, esem1, ssem0, ssem1):
        c = lax.axis_index("c")
        s = lax.axis_index("s")
        di = (di0, di1)
        ewr = (ewr0, ewr1)
        stage = (stage0, stage1)
        dsem = (dsem0, dsem1)
        esem = (esem0, esem1)
        ssem = (ssem0, ssem1)

        def zero_stage(st):
            def zr(i, carry):
                for j in range(RW // L):
                    st[i, pl.ds(j * L, L)] = jnp.zeros((L,), jnp.float32)
                return carry
            lax.fori_loop(0, CH, zr, 0)

        zero_stage(stage0)
        zero_stage(stage1)
        _for_node_chunks(s, lambda row0, rows: pltpu.sync_copy(
            stage0.at[pl.ds(0, rows)], acc_sh.at[pl.ds(row0, rows)]))
        plsc.subcore_barrier()

        sep = pl.multiple_of(s * EP, 8)
        dep = pl.multiple_of(c * EPAD, 8)

        def start(k, b):
            pltpu.async_copy(didx_h.at[pl.ds(dep + sep + k * CH, CH)], di[b],
                             dsem[b])
            pltpu.async_copy(ew16_h.at[pl.ds(sep + k * CH, CH)], ewr[b], esem[b])

        def wait_scatter(k, b):
            pltpu.make_async_copy(stage[b], acc_sh.at[di[b]], ssem[b]).wait()

        start(0, 0)

        def pair(k2, carry):
            for b in range(2):
                k = 2 * k2 + b
                b2 = 1 - b

                @pl.when(k + 1 < NCH)
                def _pre():
                    @pl.when(k >= 1)
                    def _ws():
                        wait_scatter(k - 1, b2)

                    start(k + 1, b2)

                pltpu.make_async_copy(ew16_h.at[pl.ds(sep + k * CH, CH)],
                                      ewr[b], esem[b]).wait()
                pltpu.make_async_copy(didx_h.at[pl.ds(dep + sep + k * CH, CH)],
                                      di[b], dsem[b]).wait()

                def put(i, carry2):
                    stage[b][i, pl.ds(0, L)] = ewr[b][i, :]
                    return carry2

                lax.fori_loop(0, CH, put, 0)
                pltpu.async_copy(stage[b], acc_sh.at[di[b]], ssem[b],
                                 add=True)
            return carry

        lax.fori_loop(0, NCH // 2, pair, 0)
        wait_scatter(NCH - 2, 0)
        wait_scatter(NCH - 1, 1)
        plsc.subcore_barrier()

        tbase = pl.multiple_of(c * N, 8)

        def read_out(row0, rows):
            pltpu.sync_copy(acc_sh.at[pl.ds(row0, rows)], stage0.at[pl.ds(0, rows)])
            pltpu.sync_copy(stage0.at[pl.ds(0, rows)], out_h.at[c, pl.ds(row0, rows)])
            pltpu.sync_copy(x_h.at[pl.ds(row0, rows)], stage1.at[pl.ds(0, rows)])

            def t1row(i, carry):
                degv = stage0[i, pl.ds(0, L)]
                degv = jnp.where(degv == 0.0, 1.0, degv)
                rv = 1.0 / degv
                for j in range(RW // L):
                    stage1[i, pl.ds(j * L, L)] = stage1[i, pl.ds(j * L, L)] * rv
                return carry

            lax.fori_loop(0, rows, t1row, 0)
            pltpu.sync_copy(stage1.at[pl.ds(0, rows)],
                            t1_h.at[pl.ds(tbase + row0, rows)])

        _for_node_chunks(s, read_out)

    return body(didx4, ew16, x)


def _propagate(table, gidx4, sidx4, ew, F):
    @functools.partial(
        pl.kernel,
        out_type=jax.ShapeDtypeStruct((NC, N, F), jnp.float32),
        mesh=_sc_mesh(),
        scratch_types=[
            pltpu.VMEM((CH,), jnp.int32),
            pltpu.VMEM((CH,), jnp.int32),
            pltpu.VMEM((CH,), jnp.int32),
            pltpu.VMEM((CH,), jnp.int32),
            pltpu.VMEM((CH, L), jnp.float32),
            pltpu.VMEM((CH, L), jnp.float32),
            pltpu.VMEM((CH, F), jnp.float32),
            pltpu.VMEM((CH, F), jnp.float32),
            pltpu.VMEM_SHARED((N, F), jnp.float32),
            pltpu.SemaphoreType.DMA,
            pltpu.SemaphoreType.DMA,
            pltpu.SemaphoreType.DMA,
            pltpu.SemaphoreType.DMA,
            pltpu.SemaphoreType.DMA,
            pltpu.SemaphoreType.DMA,
            pltpu.SemaphoreType.DMA,
            pltpu.SemaphoreType.DMA,
        ],
    )
    def body(tab_h, gidx_h, sidx_h, ew_h, out_h, gi0, gi1, si0, si1,
             ewr0, ewr1, rows0, rows1, acc_sh, isem0, isem1, gsem0, gsem1,
             esem0, esem1, ssem0, ssem1):
        c = lax.axis_index("c")
        s = lax.axis_index("s")
        gi = (gi0, gi1)
        si = (si0, si1)
        ewr = (ewr0, ewr1)
        rows = (rows0, rows1)
        isem = (isem0, isem1)
        gsem = (gsem0, gsem1)
        esem = (esem0, esem1)
        ssem = (ssem0, ssem1)

        def zero_rows(i, carry):
            for j in range(F // L):
                rows0[i, pl.ds(j * L, L)] = jnp.zeros((L,), jnp.float32)
            return carry

        lax.fori_loop(0, CH, zero_rows, 0)
        _for_node_chunks(s, lambda row0, rcnt: pltpu.sync_copy(
            rows0.at[pl.ds(0, rcnt)], acc_sh.at[pl.ds(row0, rcnt)]))
        plsc.subcore_barrier()

        sep = pl.multiple_of(s * EP, 8)
        cep = pl.multiple_of(c * EPAD, 8)

        def start(k, b):
            pltpu.sync_copy(gidx_h.at[pl.ds(cep + sep + k * CH, CH)], gi[b])
            pltpu.async_copy(sidx_h.at[pl.ds(cep + sep + k * CH, CH)], si[b],
                             isem[b])
            pltpu.async_copy(ew_h.at[pl.ds(sep + k * CH, CH)], ewr[b], esem[b])
            pltpu.async_copy(tab_h.at[gi[b]], rows[b], gsem[b])

        def wait_scatter(k, b):
            pltpu.make_async_copy(rows[b], acc_sh.at[si[b]], ssem[b]).wait()

        start(0, 0)

        def pair(k2, carry):
            for b in range(2):
                k = 2 * k2 + b
                b2 = 1 - b

                @pl.when(k + 1 < NCH)
                def _pre():
                    @pl.when(k >= 1)
                    def _ws():
                        wait_scatter(k - 1, b2)

                    start(k + 1, b2)

                pltpu.make_async_copy(tab_h.at[gi[b]], rows[b],
                                      gsem[b]).wait()
                pltpu.make_async_copy(ew_h.at[pl.ds(sep + k * CH, CH)],
                                      ewr[b], esem[b]).wait()
                pltpu.make_async_copy(sidx_h.at[pl.ds(cep + sep + k * CH, CH)],
                                      si[b], isem[b]).wait()

                def scale(i4, carry2):
                    for u in range(4):
                        i = i4 * 4 + u
                        w = ewr[b][i, :]
                        for j in range(F // L):
                            rows[b][i, pl.ds(j * L, L)] = (
                                rows[b][i, pl.ds(j * L, L)] * w)
                    return carry2

                lax.fori_loop(0, CH // 4, scale, 0)
                pltpu.async_copy(rows[b], acc_sh.at[si[b]], ssem[b],
                                 add=True)
            return carry

        lax.fori_loop(0, NCH // 2, pair, 0)
        wait_scatter(NCH - 2, 0)
        wait_scatter(NCH - 1, 1)
        plsc.subcore_barrier()

        def read_out(row0, rcnt):
            pltpu.sync_copy(acc_sh.at[pl.ds(row0, rcnt)], rows0.at[pl.ds(0, rcnt)])
            pltpu.sync_copy(rows0.at[pl.ds(0, rcnt)], out_h.at[c, pl.ds(row0, rcnt)])

        _for_node_chunks(s, read_out)

    return body(table, gidx4, sidx4, ew)


_BR = 1000


def _scale_table(xa, deg):
    def body(x_ref, d_ref, t_ref):
        xb = x_ref[...]
        do = d_ref[0, :, 0:1]
        di = d_ref[1, :, 0:1]
        do = jnp.where(do == 0.0, 1.0, do)
        di = jnp.where(di == 0.0, 1.0, di)
        t_ref[0] = xb / do
        t_ref[1] = xb / di

    return pl.pallas_call(
        body,
        grid=(N // _BR,),
        in_specs=[
            pl.BlockSpec((_BR, 128), lambda i: (i, 0)),
            pl.BlockSpec((2, _BR, RW), lambda i: (0, i, 0)),
        ],
        out_specs=pl.BlockSpec((2, _BR, 128), lambda i: (0, i, 0)),
        out_shape=jax.ShapeDtypeStruct((2, N, 128), jnp.float32),
    )(xa, deg)


def _layer1_dense(xa, P, deg, wz0, wzo, wzi, bz, wh0, who, whi, bh):
    def body(x_ref, p_ref, d_ref, wz0r, wzor, wzir, bzr, wh0r, whor, whir, bhr,
             h_ref, t_ref):
        xb = x_ref[...]
        p0 = p_ref[0]
        p1 = p_ref[1]
        zp = xb @ wz0r[...] + p0 @ wzor[...] + p1 @ wzir[...] + bzr[...]
        hp = xb @ wh0r[...] + p0 @ whor[...] + p1 @ whir[...] + bhr[...]
        h1 = jax.nn.relu((1.0 - jax.nn.sigmoid(zp)) * jnp.tanh(hp))
        h_ref[...] = h1
        do = d_ref[0, :, 0:1]
        di = d_ref[1, :, 0:1]
        do = jnp.where(do == 0.0, 1.0, do)
        di = jnp.where(di == 0.0, 1.0, di)
        t_ref[0] = h1 / do
        t_ref[1] = h1 / di

    full = lambda shape: pl.BlockSpec(shape, lambda i: tuple(0 for _ in shape))
    return pl.pallas_call(
        body,
        grid=(N // _BR,),
        in_specs=[
            pl.BlockSpec((_BR, 128), lambda i: (i, 0)),
            pl.BlockSpec((2, _BR, 128), lambda i: (0, i, 0)),
            pl.BlockSpec((2, _BR, RW), lambda i: (0, i, 0)),
            full((128, 128)), full((128, 128)), full((128, 128)), full((1, 128)),
            full((128, 128)), full((128, 128)), full((128, 128)), full((1, 128)),
        ],
        out_specs=[
            pl.BlockSpec((_BR, 128), lambda i: (i, 0)),
            pl.BlockSpec((2, _BR, 128), lambda i: (0, i, 0)),
        ],
        out_shape=[
            jax.ShapeDtypeStruct((N, 128), jnp.float32),
            jax.ShapeDtypeStruct((2, N, 128), jnp.float32),
        ],
    )(xa, P, deg, wz0, wzo, wzi, bz, wh0, who, whi, bh)


def _layer2_dense(h1, P2, wz0, wzo, wzi, bz, wh0, who, whi, bh, lw, lb):
    def body(h_ref, p_ref, wz0r, wzor, wzir, bzr, wh0r, whor, whir, bhr,
             lwr, lbr, o_ref):
        hb = h_ref[...]
        p0 = p_ref[0]
        p1 = p_ref[1]
        zp = hb @ wz0r[...] + p0 @ wzor[...] + p1 @ wzir[...] + bzr[...]
        hp = hb @ wh0r[...] + p0 @ whor[...] + p1 @ whir[...] + bhr[...]
        h2 = jax.nn.relu((1.0 - jax.nn.sigmoid(zp)) * jnp.tanh(hp))
        o_ref[...] = h2 @ lwr[...] + lbr[...]

    full = lambda shape: pl.BlockSpec(shape, lambda i: tuple(0 for _ in shape))
    return pl.pallas_call(
        body,
        grid=(N // _BR,),
        in_specs=[
            pl.BlockSpec((_BR, 128), lambda i: (i, 0)),
            pl.BlockSpec((2, _BR, 128), lambda i: (0, i, 0)),
            full((128, 20)), full((128, 20)), full((128, 20)), full((1, 20)),
            full((128, 20)), full((128, 20)), full((128, 20)), full((1, 20)),
            full((20, 1)), full((1, 1)),
        ],
        out_specs=pl.BlockSpec((_BR, 1), lambda i: (i, 0)),
        out_shape=jax.ShapeDtypeStruct((N, 1), jnp.float32),
    )(h1, P2, wz0, wzo, wzi, bz, wh0, who, whi, bh, lw, lb)


def _pad_cols(a, n):
    return jnp.pad(a, ((0, 0), (0, n - a.shape[1])))


def _pad_rows(a, n):
    return jnp.pad(a, ((0, n - a.shape[0]), (0, 0)))


def _pad_edges(a):
    return jnp.pad(a.reshape(NS, EPT), ((0, 0), (0, EP - EPT))).reshape(-1)


def kernel(x, edge_index, edge_weight, W1z, b1z, W1r, b1r, W1h, b1h,
           W2z, b2z, W2r, b2r, W2h, b2h, lin_W, lin_b):
    rowp = _pad_edges(edge_index[0])
    colp = _pad_edges(edge_index[1])
    ewp = _pad_edges(edge_weight)
    didx = jnp.concatenate([rowp, colp])
    gidx = jnp.concatenate([rowp, colp + N])
    sidx = jnp.concatenate([colp, rowp])
    ew_rep = jnp.broadcast_to(ewp[:, None], (EPAD, L))

    deg, T1 = _degrees(didx, ew_rep, x)
    P1 = _propagate(T1, gidx, sidx, ew_rep, 128)

    wz0 = _pad_cols(W1z[0, 0, :128] + W1z[1, 0, :128], 128)
    wzo = _pad_cols(W1z[0, 1, :128], 128)
    wzi = _pad_cols(W1z[1, 1, :128], 128)
    bzp = _pad_cols(b1z.reshape(1, 50), 128)
    wh0 = _pad_cols(W1h[0, 0, :128] + W1h[1, 0, :128], 128)
    who = _pad_cols(W1h[0, 1, :128], 128)
    whi = _pad_cols(W1h[1, 1, :128], 128)
    bhp = _pad_cols(b1h.reshape(1, 50), 128)

    h1, T2 = _layer1_dense(x, P1, deg, wz0, wzo, wzi, bzp, wh0, who, whi, bhp)
    P2 = _propagate(T2.reshape(2 * N, 128), gidx, sidx, ew_rep, 128)

    wz02 = _pad_rows(W2z[0, 0, :50] + W2z[1, 0, :50], 128)
    wzo2 = _pad_rows(W2z[0, 1, :50], 128)
    wzi2 = _pad_rows(W2z[1, 1, :50], 128)
    bz2 = b2z.reshape(1, 20)
    wh02 = _pad_rows(W2h[0, 0, :50] + W2h[1, 0, :50], 128)
    who2 = _pad_rows(W2h[0, 1, :50], 128)
    whi2 = _pad_rows(W2h[1, 1, :50], 128)
    bh2 = b2h.reshape(1, 20)

    return _layer2_dense(h1, P2, wz02, wzo2, wzi2, bz2, wh02, who2, whi2, bh2,
                         lin_W, lin_b.reshape(1, 1))


# speedup vs baseline: 1.0151x; 1.0151x over previous
import functools

import jax
import jax.numpy as jnp
from jax import lax
from jax.experimental import pallas as pl
from jax.experimental.pallas import tpu as pltpu
from jax.experimental.pallas import tpu_sc as plsc

N = 10000
E = 320000
NC = 2
NS = 16
L = 16
CH = 64
RW = 128
EPT = E // NS
EP = 2 * CH * (-(-EPT // (2 * CH)))
EPAD = NS * EP
NCH = EP // CH
NRCH = -(-N // CH)
NRQ = -(-NRCH // NS)
LASTR = N - (NRCH - 1) * CH


def _sc_mesh():
    return plsc.VectorSubcoreMesh(
        core_axis_name="c", subcore_axis_name="s", num_cores=NC, num_subcores=NS
    )


def _for_node_chunks(s, fn):
    for q in range(NRQ):
        cq = s + NS * q
        row0 = pl.multiple_of(cq * CH, 8)

        @pl.when(cq < NRCH - 1)
        def _full():
            fn(row0, CH)

        @pl.when(cq == NRCH - 1)
        def _tail():
            fn(row0, LASTR)


def _degrees(didx4, ew16, x):
    @functools.partial(
        pl.kernel,
        out_type=[
            jax.ShapeDtypeStruct((NC, N, RW), jnp.float32),
            jax.ShapeDtypeStruct((2 * N, RW), jnp.float32),
        ],
        mesh=_sc_mesh(),
        scratch_types=[
            pltpu.VMEM((CH,), jnp.int32),
            pltpu.VMEM((CH,), jnp.int32),
            pltpu.VMEM((CH, L), jnp.float32),
            pltpu.VMEM((CH, L), jnp.float32),
            pltpu.VMEM((CH, RW), jnp.float32),
            pltpu.VMEM((CH, RW), jnp.float32),
            pltpu.VMEM_SHARED((N, RW), jnp.float32),
            pltpu.SemaphoreType.DMA,
            pltpu.SemaphoreType.DMA,
            pltpu.SemaphoreType.DMA,
            pltpu.SemaphoreType.DMA,
            pltpu.SemaphoreType.DMA,
            pltpu.SemaphoreType.DMA,
        ],
    )
    def body(didx_h, ew16_h, x_h, out_h, t1_h, di0, di1, ewr0, ewr1, stage0,
             stage1, acc_sh, dsem0, dsem1, esem0, esem1, ssem0, ssem1):
        c = lax.axis_index("c")
        s = lax.axis_index("s")
        di = (di0, di1)
        ewr = (ewr0, ewr1)
        stage = (stage0, stage1)
        dsem = (dsem0, dsem1)
        esem = (esem0, esem1)
        ssem = (ssem0, ssem1)

        def zero_stage(st):
            def zr(i, carry):
                for j in range(RW // L):
                    st[i, pl.ds(j * L, L)] = jnp.zeros((L,), jnp.float32)
                return carry
            lax.fori_loop(0, CH, zr, 0)

        zero_stage(stage0)
        zero_stage(stage1)
        _for_node_chunks(s, lambda row0, rows: pltpu.sync_copy(
            stage0.at[pl.ds(0, rows)], acc_sh.at[pl.ds(row0, rows)]))
        plsc.subcore_barrier()

        sep = pl.multiple_of(s * EP, 8)
        dep = pl.multiple_of(c * EPAD, 8)

        def start(k, b):
            pltpu.async_copy(didx_h.at[pl.ds(dep + sep + k * CH, CH)], di[b],
                             dsem[b])
            pltpu.async_copy(ew16_h.at[pl.ds(sep + k * CH, CH)], ewr[b], esem[b])

        def wait_scatter(k, b):
            pltpu.make_async_copy(stage[b], acc_sh.at[di[b]], ssem[b]).wait()

        start(0, 0)

        def pair(k2, carry):
            for b in range(2):
                k = 2 * k2 + b
                b2 = 1 - b

                @pl.when(k + 1 < NCH)
                def _pre():
                    @pl.when(k >= 1)
                    def _ws():
                        wait_scatter(k - 1, b2)

                    start(k + 1, b2)

                pltpu.make_async_copy(ew16_h.at[pl.ds(sep + k * CH, CH)],
                                      ewr[b], esem[b]).wait()
                pltpu.make_async_copy(didx_h.at[pl.ds(dep + sep + k * CH, CH)],
                                      di[b], dsem[b]).wait()

                def put(i8, carry2):
                    for u in range(8):
                        i = i8 * 8 + u
                        stage[b][i, pl.ds(0, L)] = ewr[b][i, :]
                    return carry2

                lax.fori_loop(0, CH // 8, put, 0)
                pltpu.async_copy(stage[b], acc_sh.at[di[b]], ssem[b],
                                 add=True)
            return carry

        lax.fori_loop(0, NCH // 2, pair, 0)
        wait_scatter(NCH - 2, 0)
        wait_scatter(NCH - 1, 1)
        plsc.subcore_barrier()

        tbase = pl.multiple_of(c * N, 8)

        def read_out(row0, rows):
            pltpu.sync_copy(acc_sh.at[pl.ds(row0, rows)], stage0.at[pl.ds(0, rows)])
            pltpu.sync_copy(stage0.at[pl.ds(0, rows)], out_h.at[c, pl.ds(row0, rows)])
            pltpu.sync_copy(x_h.at[pl.ds(row0, rows)], stage1.at[pl.ds(0, rows)])

            def t1row(i, carry):
                degv = stage0[i, pl.ds(0, L)]
                degv = jnp.where(degv == 0.0, 1.0, degv)
                rv = 1.0 / degv
                for j in range(RW // L):
                    stage1[i, pl.ds(j * L, L)] = stage1[i, pl.ds(j * L, L)] * rv
                return carry

            lax.fori_loop(0, rows, t1row, 0)
            pltpu.sync_copy(stage1.at[pl.ds(0, rows)],
                            t1_h.at[pl.ds(tbase + row0, rows)])

        _for_node_chunks(s, read_out)

    return body(didx4, ew16, x)


def _propagate(table, gidx4, sidx4, ew, F):
    @functools.partial(
        pl.kernel,
        out_type=jax.ShapeDtypeStruct((NC, N, F), jnp.float32),
        mesh=_sc_mesh(),
        scratch_types=[
            pltpu.VMEM((CH,), jnp.int32),
            pltpu.VMEM((CH,), jnp.int32),
            pltpu.VMEM((CH,), jnp.int32),
            pltpu.VMEM((CH,), jnp.int32),
            pltpu.VMEM((CH, L), jnp.float32),
            pltpu.VMEM((CH, L), jnp.float32),
            pltpu.VMEM((CH, F), jnp.float32),
            pltpu.VMEM((CH, F), jnp.float32),
            pltpu.VMEM_SHARED((N, F), jnp.float32),
            pltpu.SemaphoreType.DMA,
            pltpu.SemaphoreType.DMA,
            pltpu.SemaphoreType.DMA,
            pltpu.SemaphoreType.DMA,
            pltpu.SemaphoreType.DMA,
            pltpu.SemaphoreType.DMA,
            pltpu.SemaphoreType.DMA,
            pltpu.SemaphoreType.DMA,
        ],
    )
    def body(tab_h, gidx_h, sidx_h, ew_h, out_h, gi0, gi1, si0, si1,
             ewr0, ewr1, rows0, rows1, acc_sh, isem0, isem1, gsem0, gsem1,
             esem0, esem1, ssem0, ssem1):
        c = lax.axis_index("c")
        s = lax.axis_index("s")
        gi = (gi0, gi1)
        si = (si0, si1)
        ewr = (ewr0, ewr1)
        rows = (rows0, rows1)
        isem = (isem0, isem1)
        gsem = (gsem0, gsem1)
        esem = (esem0, esem1)
        ssem = (ssem0, ssem1)

        def zero_rows(i, carry):
            for j in range(F // L):
                rows0[i, pl.ds(j * L, L)] = jnp.zeros((L,), jnp.float32)
            return carry

        lax.fori_loop(0, CH, zero_rows, 0)
        _for_node_chunks(s, lambda row0, rcnt: pltpu.sync_copy(
            rows0.at[pl.ds(0, rcnt)], acc_sh.at[pl.ds(row0, rcnt)]))
        plsc.subcore_barrier()

        sep = pl.multiple_of(s * EP, 8)
        cep = pl.multiple_of(c * EPAD, 8)

        def start(k, b):
            pltpu.sync_copy(gidx_h.at[pl.ds(cep + sep + k * CH, CH)], gi[b])
            pltpu.async_copy(sidx_h.at[pl.ds(cep + sep + k * CH, CH)], si[b],
                             isem[b])
            pltpu.async_copy(ew_h.at[pl.ds(sep + k * CH, CH)], ewr[b], esem[b])
            pltpu.async_copy(tab_h.at[gi[b]], rows[b], gsem[b])

        def wait_scatter(k, b):
            pltpu.make_async_copy(rows[b], acc_sh.at[si[b]], ssem[b]).wait()

        start(0, 0)

        def pair(k2, carry):
            for b in range(2):
                k = 2 * k2 + b
                b2 = 1 - b

                @pl.when(k + 1 < NCH)
                def _pre():
                    @pl.when(k >= 1)
                    def _ws():
                        wait_scatter(k - 1, b2)

                    start(k + 1, b2)

                pltpu.make_async_copy(tab_h.at[gi[b]], rows[b],
                                      gsem[b]).wait()
                pltpu.make_async_copy(ew_h.at[pl.ds(sep + k * CH, CH)],
                                      ewr[b], esem[b]).wait()
                pltpu.make_async_copy(sidx_h.at[pl.ds(cep + sep + k * CH, CH)],
                                      si[b], isem[b]).wait()

                def scale(i8, carry2):
                    for u in range(8):
                        i = i8 * 8 + u
                        w = ewr[b][i, :]
                        for j in range(F // L):
                            rows[b][i, pl.ds(j * L, L)] = (
                                rows[b][i, pl.ds(j * L, L)] * w)
                    return carry2

                lax.fori_loop(0, CH // 8, scale, 0)
                pltpu.async_copy(rows[b], acc_sh.at[si[b]], ssem[b],
                                 add=True)
            return carry

        lax.fori_loop(0, NCH // 2, pair, 0)
        wait_scatter(NCH - 2, 0)
        wait_scatter(NCH - 1, 1)
        plsc.subcore_barrier()

        def read_out(row0, rcnt):
            pltpu.sync_copy(acc_sh.at[pl.ds(row0, rcnt)], rows0.at[pl.ds(0, rcnt)])
            pltpu.sync_copy(rows0.at[pl.ds(0, rcnt)], out_h.at[c, pl.ds(row0, rcnt)])

        _for_node_chunks(s, read_out)

    return body(table, gidx4, sidx4, ew)


_BR = 1000


def _full_spec(shape):
    return pl.BlockSpec(shape, lambda i, _s=len(shape): (0,) * _s)


def _xw1(xa, wz0, bz, wh0, bh):
    def body(x_ref, wz0r, bzr, wh0r, bhr, xz_ref, xh_ref):
        xb = x_ref[...]
        xz_ref[...] = xb @ wz0r[...] + bzr[...]
        xh_ref[...] = xb @ wh0r[...] + bhr[...]

    return pl.pallas_call(
        body,
        grid=(N // _BR,),
        in_specs=[
            pl.BlockSpec((_BR, 128), lambda i: (i, 0)),
            _full_spec((128, 128)), _full_spec((1, 128)),
            _full_spec((128, 128)), _full_spec((1, 128)),
        ],
        out_specs=[
            pl.BlockSpec((_BR, 128), lambda i: (i, 0)),
            pl.BlockSpec((_BR, 128), lambda i: (i, 0)),
        ],
        out_shape=[
            jax.ShapeDtypeStruct((N, 128), jnp.float32),
            jax.ShapeDtypeStruct((N, 128), jnp.float32),
        ],
    )(xa, wz0, bz, wh0, bh)


def _layer1_dense(xwz, xwh, P, deg, wzo, wzi, who, whi, wz02, bz2, wh02, bh2):
    def body(xz_ref, xh_ref, p_ref, d_ref, wzor, wzir, whor, whir,
             wz02r, bz2r, wh02r, bh2r, t_ref, x2z_ref, x2h_ref):
        p0 = p_ref[0]
        p1 = p_ref[1]
        zp = xz_ref[...] + p0 @ wzor[...] + p1 @ wzir[...]
        hp = xh_ref[...] + p0 @ whor[...] + p1 @ whir[...]
        h1 = jax.nn.relu((1.0 - jax.nn.sigmoid(zp)) * jnp.tanh(hp))
        do = d_ref[0, :, 0:1]
        di = d_ref[1, :, 0:1]
        do = jnp.where(do == 0.0, 1.0, do)
        di = jnp.where(di == 0.0, 1.0, di)
        t_ref[0] = h1 / do
        t_ref[1] = h1 / di
        x2z_ref[...] = h1 @ wz02r[...] + bz2r[...]
        x2h_ref[...] = h1 @ wh02r[...] + bh2r[...]

    return pl.pallas_call(
        body,
        grid=(N // _BR,),
        in_specs=[
            pl.BlockSpec((_BR, 128), lambda i: (i, 0)),
            pl.BlockSpec((_BR, 128), lambda i: (i, 0)),
            pl.BlockSpec((2, _BR, 128), lambda i: (0, i, 0)),
            pl.BlockSpec((2, _BR, RW), lambda i: (0, i, 0)),
            _full_spec((128, 128)), _full_spec((128, 128)),
            _full_spec((128, 128)), _full_spec((128, 128)),
            _full_spec((128, 20)), _full_spec((1, 20)),
            _full_spec((128, 20)), _full_spec((1, 20)),
        ],
        out_specs=[
            pl.BlockSpec((2, _BR, 128), lambda i: (0, i, 0)),
            pl.BlockSpec((_BR, 20), lambda i: (i, 0)),
            pl.BlockSpec((_BR, 20), lambda i: (i, 0)),
        ],
        out_shape=[
            jax.ShapeDtypeStruct((2, N, 128), jnp.float32),
            jax.ShapeDtypeStruct((N, 20), jnp.float32),
            jax.ShapeDtypeStruct((N, 20), jnp.float32),
        ],
    )(xwz, xwh, P, deg, wzo, wzi, who, whi, wz02, bz2, wh02, bh2)


def _layer2_dense(x2z, x2h, P2, wzo, wzi, who, whi, lw, lb):
    def body(xz_ref, xh_ref, p_ref, wzor, wzir, whor, whir, lwr, lbr, o_ref):
        p0 = p_ref[0]
        p1 = p_ref[1]
        zp = xz_ref[...] + p0 @ wzor[...] + p1 @ wzir[...]
        hp = xh_ref[...] + p0 @ whor[...] + p1 @ whir[...]
        h2 = jax.nn.relu((1.0 - jax.nn.sigmoid(zp)) * jnp.tanh(hp))
        o_ref[...] = h2 @ lwr[...] + lbr[...]

    return pl.pallas_call(
        body,
        grid=(N // _BR,),
        in_specs=[
            pl.BlockSpec((_BR, 20), lambda i: (i, 0)),
            pl.BlockSpec((_BR, 20), lambda i: (i, 0)),
            pl.BlockSpec((2, _BR, 128), lambda i: (0, i, 0)),
            _full_spec((128, 20)), _full_spec((128, 20)),
            _full_spec((128, 20)), _full_spec((128, 20)),
            _full_spec((20, 1)), _full_spec((1, 1)),
        ],
        out_specs=pl.BlockSpec((_BR, 1), lambda i: (i, 0)),
        out_shape=jax.ShapeDtypeStruct((N, 1), jnp.float32),
    )(x2z, x2h, P2, wzo, wzi, who, whi, lw, lb)


def _pad_cols(a, n):
    return jnp.pad(a, ((0, 0), (0, n - a.shape[1])))


def _pad_rows(a, n):
    return jnp.pad(a, ((0, n - a.shape[0]), (0, 0)))


def _pad_edges(a):
    return jnp.pad(a.reshape(NS, EPT), ((0, 0), (0, EP - EPT))).reshape(-1)


def kernel(x, edge_index, edge_weight, W1z, b1z, W1r, b1r, W1h, b1h,
           W2z, b2z, W2r, b2r, W2h, b2h, lin_W, lin_b):
    rowp = _pad_edges(edge_index[0])
    colp = _pad_edges(edge_index[1])
    ewp = _pad_edges(edge_weight)
    didx = jnp.concatenate([rowp, colp])
    gidx = jnp.concatenate([rowp, colp + N])
    sidx = jnp.concatenate([colp, rowp])
    ew_rep = jnp.broadcast_to(ewp[:, None], (EPAD, L))

    wz0 = _pad_cols(W1z[0, 0, :128] + W1z[1, 0, :128], 128)
    wzo = _pad_cols(W1z[0, 1, :128], 128)
    wzi = _pad_cols(W1z[1, 1, :128], 128)
    bzp = _pad_cols(b1z.reshape(1, 50), 128)
    wh0 = _pad_cols(W1h[0, 0, :128] + W1h[1, 0, :128], 128)
    who = _pad_cols(W1h[0, 1, :128], 128)
    whi = _pad_cols(W1h[1, 1, :128], 128)
    bhp = _pad_cols(b1h.reshape(1, 50), 128)
    wz02 = _pad_rows(W2z[0, 0, :50] + W2z[1, 0, :50], 128)
    wzo2 = _pad_rows(W2z[0, 1, :50], 128)
    wzi2 = _pad_rows(W2z[1, 1, :50], 128)
    bz2 = b2z.reshape(1, 20)
    wh02 = _pad_rows(W2h[0, 0, :50] + W2h[1, 0, :50], 128)
    who2 = _pad_rows(W2h[0, 1, :50], 128)
    whi2 = _pad_rows(W2h[1, 1, :50], 128)
    bh2 = b2h.reshape(1, 20)

    deg, T1 = _degrees(didx, ew_rep, x)
    xwz, xwh = _xw1(x, wz0, bzp, wh0, bhp)
    P1 = _propagate(T1, gidx, sidx, ew_rep, 128)
    T2, x2z, x2h = _layer1_dense(xwz, xwh, P1, deg, wzo, wzi, who, whi,
                                 wz02, bz2, wh02, bh2)
    P2 = _propagate(T2.reshape(2 * N, 128), gidx, sidx, ew_rep, 128)
    return _layer2_dense(x2z, x2h, P2, wzo2, wzi2, who2, whi2,
                         lin_W, lin_b.reshape(1, 1))


# speedup vs baseline: 1.0904x; 1.0742x over previous
import functools

import jax
import jax.numpy as jnp
from jax import lax
from jax.experimental import pallas as pl
from jax.experimental.pallas import tpu as pltpu
from jax.experimental.pallas import tpu_sc as plsc

N = 10000
E = 320000
NC = 2
NS = 16
L = 16
CH = 64
RW = 128
EPT = E // NS
EP = 2 * CH * (-(-EPT // (2 * CH)))
EPAD = NS * EP
NCH = EP // CH
NRCH = -(-N // CH)
NRQ = -(-NRCH // NS)
LASTR = N - (NRCH - 1) * CH


def _sc_mesh():
    return plsc.VectorSubcoreMesh(
        core_axis_name="c", subcore_axis_name="s", num_cores=NC, num_subcores=NS
    )


def _for_node_chunks(s, fn):
    for q in range(NRQ):
        cq = s + NS * q
        row0 = pl.multiple_of(cq * CH, 8)

        @pl.when(cq < NRCH - 1)
        def _full():
            fn(row0, CH)

        @pl.when(cq == NRCH - 1)
        def _tail():
            fn(row0, LASTR)


def _degrees(didx4, ew16, x):
    @functools.partial(
        pl.kernel,
        out_type=[
            jax.ShapeDtypeStruct((NC, N, RW), jnp.float32),
            jax.ShapeDtypeStruct((2 * N, RW), jnp.float32),
        ],
        mesh=_sc_mesh(),
        scratch_types=[
            pltpu.VMEM((CH,), jnp.int32),
            pltpu.VMEM((CH,), jnp.int32),
            pltpu.VMEM((CH, L), jnp.float32),
            pltpu.VMEM((CH, L), jnp.float32),
            pltpu.VMEM((CH, RW), jnp.float32),
            pltpu.VMEM((CH, RW), jnp.float32),
            pltpu.VMEM_SHARED((N, RW), jnp.float32),
            pltpu.SemaphoreType.DMA,
            pltpu.SemaphoreType.DMA,
            pltpu.SemaphoreType.DMA,
            pltpu.SemaphoreType.DMA,
            pltpu.SemaphoreType.DMA,
            pltpu.SemaphoreType.DMA,
        ],
    )
    def body(didx_h, ew16_h, x_h, out_h, t1_h, di0, di1, ewr0, ewr1, stage0,
             stage1, acc_sh, dsem0, dsem1, esem0, esem1, ssem0, ssem1):
        c = lax.axis_index("c")
        s = lax.axis_index("s")
        di = (di0, di1)
        ewr = (ewr0, ewr1)
        stage = (stage0, stage1)
        dsem = (dsem0, dsem1)
        esem = (esem0, esem1)
        ssem = (ssem0, ssem1)

        def zero_stage(st):
            def zr(i, carry):
                for j in range(RW // L):
                    st[i, pl.ds(j * L, L)] = jnp.zeros((L,), jnp.float32)
                return carry
            lax.fori_loop(0, CH, zr, 0)

        zero_stage(stage0)
        zero_stage(stage1)
        _for_node_chunks(s, lambda row0, rows: pltpu.sync_copy(
            stage0.at[pl.ds(0, rows)], acc_sh.at[pl.ds(row0, rows)]))
        plsc.subcore_barrier()

        sep = pl.multiple_of(s * EP, 8)
        dep = pl.multiple_of(c * EPAD, 8)

        def start(k, b):
            pltpu.async_copy(didx_h.at[pl.ds(dep + sep + k * CH, CH)], di[b],
                             dsem[b])
            pltpu.async_copy(ew16_h.at[pl.ds(sep + k * CH, CH)], ewr[b], esem[b])

        def wait_scatter(k, b):
            pltpu.make_async_copy(stage[b], acc_sh.at[di[b]], ssem[b]).wait()

        start(0, 0)

        def pair(k2, carry):
            for b in range(2):
                k = 2 * k2 + b
                b2 = 1 - b

                @pl.when(k + 1 < NCH)
                def _pre():
                    @pl.when(k >= 1)
                    def _ws():
                        wait_scatter(k - 1, b2)

                    start(k + 1, b2)

                pltpu.make_async_copy(ew16_h.at[pl.ds(sep + k * CH, CH)],
                                      ewr[b], esem[b]).wait()
                pltpu.make_async_copy(didx_h.at[pl.ds(dep + sep + k * CH, CH)],
                                      di[b], dsem[b]).wait()

                def put(i8, carry2):
                    for u in range(8):
                        i = i8 * 8 + u
                        stage[b][i, pl.ds(0, L)] = ewr[b][i, :]
                    return carry2

                lax.fori_loop(0, CH // 8, put, 0)
                pltpu.async_copy(stage[b], acc_sh.at[di[b]], ssem[b],
                                 add=True)
            return carry

        lax.fori_loop(0, NCH // 2, pair, 0)
        wait_scatter(NCH - 2, 0)
        wait_scatter(NCH - 1, 1)
        plsc.subcore_barrier()

        tbase = pl.multiple_of(c * N, 8)

        def read_out(row0, rows):
            pltpu.sync_copy(acc_sh.at[pl.ds(row0, rows)], stage0.at[pl.ds(0, rows)])
            pltpu.sync_copy(stage0.at[pl.ds(0, rows)], out_h.at[c, pl.ds(row0, rows)])
            pltpu.sync_copy(x_h.at[pl.ds(row0, rows)], stage1.at[pl.ds(0, rows)])

            def t1row(i, carry):
                degv = stage0[i, pl.ds(0, L)]
                degv = jnp.where(degv == 0.0, 1.0, degv)
                rv = 1.0 / degv
                for j in range(RW // L):
                    stage1[i, pl.ds(j * L, L)] = stage1[i, pl.ds(j * L, L)] * rv
                return carry

            lax.fori_loop(0, rows, t1row, 0)
            pltpu.sync_copy(stage1.at[pl.ds(0, rows)],
                            t1_h.at[pl.ds(tbase + row0, rows)])

        _for_node_chunks(s, read_out)

    return body(didx4, ew16, x)


def _propagate(table, gidx4, sidx4, ew, F):
    @functools.partial(
        pl.kernel,
        out_type=jax.ShapeDtypeStruct((NC, N, F), jnp.float32),
        mesh=_sc_mesh(),
        scratch_types=[
            pltpu.VMEM((CH,), jnp.int32),
            pltpu.VMEM((CH,), jnp.int32),
            pltpu.VMEM((CH,), jnp.int32),
            pltpu.VMEM((CH,), jnp.int32),
            pltpu.VMEM((CH, L), jnp.float32),
            pltpu.VMEM((CH, L), jnp.float32),
            pltpu.VMEM((CH, F), jnp.float32),
            pltpu.VMEM((CH, F), jnp.float32),
            pltpu.VMEM_SHARED((N, F), jnp.float32),
            pltpu.SemaphoreType.DMA,
            pltpu.SemaphoreType.DMA,
            pltpu.SemaphoreType.DMA,
            pltpu.SemaphoreType.DMA,
            pltpu.SemaphoreType.DMA,
            pltpu.SemaphoreType.DMA,
            pltpu.SemaphoreType.DMA,
            pltpu.SemaphoreType.DMA,
            pltpu.SemaphoreType.DMA,
            pltpu.SemaphoreType.DMA,
        ],
    )
    def body(tab_h, gidx_h, sidx_h, ew_h, out_h, gi0, gi1, si0, si1,
             ewr0, ewr1, rows0, rows1, acc_sh, isem0, isem1, gsem0, gsem1,
             esem0, esem1, ssem0, ssem1, g2sem0, g2sem1):
        c = lax.axis_index("c")
        s = lax.axis_index("s")
        gi = (gi0, gi1)
        si = (si0, si1)
        ewr = (ewr0, ewr1)
        rows = (rows0, rows1)
        isem = (isem0, isem1)
        gsem = (gsem0, gsem1)
        esem = (esem0, esem1)
        ssem = (ssem0, ssem1)
        g2sem = (g2sem0, g2sem1)

        def zero_rows(i, carry):
            for j in range(F // L):
                rows0[i, pl.ds(j * L, L)] = jnp.zeros((L,), jnp.float32)
            return carry

        lax.fori_loop(0, CH, zero_rows, 0)
        _for_node_chunks(s, lambda row0, rcnt: pltpu.sync_copy(
            rows0.at[pl.ds(0, rcnt)], acc_sh.at[pl.ds(row0, rcnt)]))
        plsc.subcore_barrier()

        sep = pl.multiple_of(s * EP, 8)
        cep = pl.multiple_of(c * EPAD, 8)

        def start_gidx(k, b):
            pltpu.async_copy(gidx_h.at[pl.ds(cep + sep + k * CH, CH)], gi[b],
                             g2sem[b])

        def start(k, b):
            pltpu.make_async_copy(gidx_h.at[pl.ds(cep + sep + k * CH, CH)],
                                  gi[b], g2sem[b]).wait()
            pltpu.async_copy(sidx_h.at[pl.ds(cep + sep + k * CH, CH)], si[b],
                             isem[b])
            pltpu.async_copy(ew_h.at[pl.ds(sep + k * CH, CH)], ewr[b], esem[b])
            pltpu.async_copy(tab_h.at[gi[b]], rows[b], gsem[b])

        def wait_scatter(k, b):
            pltpu.make_async_copy(rows[b], acc_sh.at[si[b]], ssem[b]).wait()

        start_gidx(0, 0)
        start_gidx(1, 1)
        start(0, 0)

        def pair(k2, carry):
            for b in range(2):
                k = 2 * k2 + b
                b2 = 1 - b

                @pl.when(k + 1 < NCH)
                def _pre():
                    @pl.when(k >= 1)
                    def _ws():
                        wait_scatter(k - 1, b2)

                    start(k + 1, b2)

                pltpu.make_async_copy(tab_h.at[gi[b]], rows[b],
                                      gsem[b]).wait()
                pltpu.make_async_copy(ew_h.at[pl.ds(sep + k * CH, CH)],
                                      ewr[b], esem[b]).wait()
                pltpu.make_async_copy(sidx_h.at[pl.ds(cep + sep + k * CH, CH)],
                                      si[b], isem[b]).wait()

                @pl.when(k + 2 < NCH)
                def _gpre():
                    start_gidx(k + 2, b)

                def scale(i8, carry2):
                    for u in range(8):
                        i = i8 * 8 + u
                        w = ewr[b][i, :]
                        for j in range(F // L):
                            rows[b][i, pl.ds(j * L, L)] = (
                                rows[b][i, pl.ds(j * L, L)] * w)
                    return carry2

                lax.fori_loop(0, CH // 8, scale, 0)
                pltpu.async_copy(rows[b], acc_sh.at[si[b]], ssem[b],
                                 add=True)
            return carry

        lax.fori_loop(0, NCH // 2, pair, 0)
        wait_scatter(NCH - 2, 0)
        wait_scatter(NCH - 1, 1)
        plsc.subcore_barrier()

        def read_out(row0, rcnt):
            pltpu.sync_copy(acc_sh.at[pl.ds(row0, rcnt)], rows0.at[pl.ds(0, rcnt)])
            pltpu.sync_copy(rows0.at[pl.ds(0, rcnt)], out_h.at[c, pl.ds(row0, rcnt)])

        _for_node_chunks(s, read_out)

    return body(table, gidx4, sidx4, ew)


_BR = 1000


def _full_spec(shape):
    return pl.BlockSpec(shape, lambda i, _s=len(shape): (0,) * _s)


def _xw1(xa, wz0, bz, wh0, bh):
    def body(x_ref, wz0r, bzr, wh0r, bhr, xz_ref, xh_ref):
        xb = x_ref[...]
        xz_ref[...] = xb @ wz0r[...] + bzr[...]
        xh_ref[...] = xb @ wh0r[...] + bhr[...]

    return pl.pallas_call(
        body,
        grid=(N // _BR,),
        in_specs=[
            pl.BlockSpec((_BR, 128), lambda i: (i, 0)),
            _full_spec((128, 128)), _full_spec((1, 128)),
            _full_spec((128, 128)), _full_spec((1, 128)),
        ],
        out_specs=[
            pl.BlockSpec((_BR, 128), lambda i: (i, 0)),
            pl.BlockSpec((_BR, 128), lambda i: (i, 0)),
        ],
        out_shape=[
            jax.ShapeDtypeStruct((N, 128), jnp.float32),
            jax.ShapeDtypeStruct((N, 128), jnp.float32),
        ],
    )(xa, wz0, bz, wh0, bh)


def _layer1_dense(xwz, xwh, P, deg, wzo, wzi, who, whi, wz02, bz2, wh02, bh2):
    def body(xz_ref, xh_ref, p_ref, d_ref, wzor, wzir, whor, whir,
             wz02r, bz2r, wh02r, bh2r, t_ref, x2z_ref, x2h_ref):
        p0 = p_ref[0]
        p1 = p_ref[1]
        zp = xz_ref[...] + p0 @ wzor[...] + p1 @ wzir[...]
        hp = xh_ref[...] + p0 @ whor[...] + p1 @ whir[...]
        h1 = jax.nn.relu((1.0 - jax.nn.sigmoid(zp)) * jnp.tanh(hp))
        do = d_ref[0, :, 0:1]
        di = d_ref[1, :, 0:1]
        do = jnp.where(do == 0.0, 1.0, do)
        di = jnp.where(di == 0.0, 1.0, di)
        t_ref[0] = h1 / do
        t_ref[1] = h1 / di
        x2z_ref[...] = h1 @ wz02r[...] + bz2r[...]
        x2h_ref[...] = h1 @ wh02r[...] + bh2r[...]

    return pl.pallas_call(
        body,
        grid=(N // _BR,),
        in_specs=[
            pl.BlockSpec((_BR, 128), lambda i: (i, 0)),
            pl.BlockSpec((_BR, 128), lambda i: (i, 0)),
            pl.BlockSpec((2, _BR, 128), lambda i: (0, i, 0)),
            pl.BlockSpec((2, _BR, RW), lambda i: (0, i, 0)),
            _full_spec((128, 128)), _full_spec((128, 128)),
            _full_spec((128, 128)), _full_spec((128, 128)),
            _full_spec((128, 20)), _full_spec((1, 20)),
            _full_spec((128, 20)), _full_spec((1, 20)),
        ],
        out_specs=[
            pl.BlockSpec((2, _BR, 128), lambda i: (0, i, 0)),
            pl.BlockSpec((_BR, 20), lambda i: (i, 0)),
            pl.BlockSpec((_BR, 20), lambda i: (i, 0)),
        ],
        out_shape=[
            jax.ShapeDtypeStruct((2, N, 128), jnp.float32),
            jax.ShapeDtypeStruct((N, 20), jnp.float32),
            jax.ShapeDtypeStruct((N, 20), jnp.float32),
        ],
    )(xwz, xwh, P, deg, wzo, wzi, who, whi, wz02, bz2, wh02, bh2)


def _layer2_dense(x2z, x2h, P2, wzo, wzi, who, whi, lw, lb):
    def body(xz_ref, xh_ref, p_ref, wzor, wzir, whor, whir, lwr, lbr, o_ref):
        p0 = p_ref[0]
        p1 = p_ref[1]
        zp = xz_ref[...] + p0 @ wzor[...] + p1 @ wzir[...]
        hp = xh_ref[...] + p0 @ whor[...] + p1 @ whir[...]
        h2 = jax.nn.relu((1.0 - jax.nn.sigmoid(zp)) * jnp.tanh(hp))
        o_ref[...] = h2 @ lwr[...] + lbr[...]

    return pl.pallas_call(
        body,
        grid=(N // _BR,),
        in_specs=[
            pl.BlockSpec((_BR, 20), lambda i: (i, 0)),
            pl.BlockSpec((_BR, 20), lambda i: (i, 0)),
            pl.BlockSpec((2, _BR, 128), lambda i: (0, i, 0)),
            _full_spec((128, 20)), _full_spec((128, 20)),
            _full_spec((128, 20)), _full_spec((128, 20)),
            _full_spec((20, 1)), _full_spec((1, 1)),
        ],
        out_specs=pl.BlockSpec((_BR, 1), lambda i: (i, 0)),
        out_shape=jax.ShapeDtypeStruct((N, 1), jnp.float32),
    )(x2z, x2h, P2, wzo, wzi, who, whi, lw, lb)


def _pad_cols(a, n):
    return jnp.pad(a, ((0, 0), (0, n - a.shape[1])))


def _pad_rows(a, n):
    return jnp.pad(a, ((0, n - a.shape[0]), (0, 0)))


def _pad_edges(a):
    return jnp.pad(a.reshape(NS, EPT), ((0, 0), (0, EP - EPT))).reshape(-1)


def kernel(x, edge_index, edge_weight, W1z, b1z, W1r, b1r, W1h, b1h,
           W2z, b2z, W2r, b2r, W2h, b2h, lin_W, lin_b):
    rowp = _pad_edges(edge_index[0])
    colp = _pad_edges(edge_index[1])
    ewp = _pad_edges(edge_weight)
    didx = jnp.concatenate([rowp, colp])
    gidx = jnp.concatenate([rowp, colp + N])
    sidx = jnp.concatenate([colp, rowp])
    ew_rep = jnp.broadcast_to(ewp[:, None], (EPAD, L))

    wz0 = _pad_cols(W1z[0, 0, :128] + W1z[1, 0, :128], 128)
    wzo = _pad_cols(W1z[0, 1, :128], 128)
    wzi = _pad_cols(W1z[1, 1, :128], 128)
    bzp = _pad_cols(b1z.reshape(1, 50), 128)
    wh0 = _pad_cols(W1h[0, 0, :128] + W1h[1, 0, :128], 128)
    who = _pad_cols(W1h[0, 1, :128], 128)
    whi = _pad_cols(W1h[1, 1, :128], 128)
    bhp = _pad_cols(b1h.reshape(1, 50), 128)
    wz02 = _pad_rows(W2z[0, 0, :50] + W2z[1, 0, :50], 128)
    wzo2 = _pad_rows(W2z[0, 1, :50], 128)
    wzi2 = _pad_rows(W2z[1, 1, :50], 128)
    bz2 = b2z.reshape(1, 20)
    wh02 = _pad_rows(W2h[0, 0, :50] + W2h[1, 0, :50], 128)
    who2 = _pad_rows(W2h[0, 1, :50], 128)
    whi2 = _pad_rows(W2h[1, 1, :50], 128)
    bh2 = b2h.reshape(1, 20)

    deg, T1 = _degrees(didx, ew_rep, x)
    xwz, xwh = _xw1(x, wz0, bzp, wh0, bhp)
    P1 = _propagate(T1, gidx, sidx, ew_rep, 128)
    T2, x2z, x2h = _layer1_dense(xwz, xwh, P1, deg, wzo, wzi, who, whi,
                                 wz02, bz2, wh02, bh2)
    P2 = _propagate(T2.reshape(2 * N, 128), gidx, sidx, ew_rep, 128)
    return _layer2_dense(x2z, x2h, P2, wzo2, wzi2, who2, whi2,
                         lin_W, lin_b.reshape(1, 1))
